# baseline (speedup 1.0000x reference)
import functools

import jax
import jax.numpy as jnp
from jax import lax
from jax.experimental import pallas as pl
from jax.experimental.pallas import tpu as pltpu
from jax.experimental.pallas import tpu_sc as plsc

VOCAB = 1000000
DIM = 64
BATCH = 4096
HIST = 200

NC = 2
NS = 16
NW = NC * NS
TOTAL = BATCH * HIST
BPW = TOTAL // NW
CHUNK = 128
NCHUNK = BPW // CHUNK

_mesh = plsc.VectorSubcoreMesh(core_axis_name="c", subcore_axis_name="s")


@functools.partial(
    pl.kernel,
    out_type=jax.ShapeDtypeStruct((TOTAL, DIM), jnp.float32),
    mesh=_mesh,
    scratch_types=[
        pltpu.VMEM((NCHUNK, CHUNK), jnp.int32),
        pltpu.VMEM((CHUNK, DIM), jnp.float32),
        pltpu.VMEM((CHUNK, DIM), jnp.float32),
        pltpu.SemaphoreType.DMA,
        pltpu.SemaphoreType.DMA,
    ],
    compiler_params=pltpu.CompilerParams(use_tc_tiling_on_sc=False),
)
def _sc_gather(x_hbm, table_hbm, out_hbm, idx_v, rows0, rows1, sem0, sem1):
    wid = lax.axis_index("s") * NC + lax.axis_index("c")
    base = wid * BPW
    pltpu.sync_copy(x_hbm.at[wid], idx_v)

    bufs = (rows0, rows1)
    sems = (sem0, sem1)

    pltpu.async_copy(table_hbm.at[idx_v.at[0]], bufs[0], sems[0])

    def body(j, _):
        slot = lax.rem(j, 2)

        def step(k):
            @pl.when(j + 1 < NCHUNK)
            def _():
                pltpu.async_copy(
                    table_hbm.at[idx_v.at[j + 1]], bufs[1 - k], sems[1 - k]
                )

            pltpu.make_async_copy(
                table_hbm.at[idx_v.at[j]], bufs[k], sems[k]
            ).wait()
            pltpu.sync_copy(bufs[k], out_hbm.at[pl.ds(base + j * CHUNK, CHUNK)])

        @pl.when(slot == 0)
        def _():
            step(0)

        @pl.when(slot == 1)
        def _():
            step(1)

        return 0

    lax.fori_loop(0, NCHUNK, body, 0, unroll=False)


def kernel(x, table):
    x_flat = x.astype(jnp.int32).reshape(NW, NCHUNK, CHUNK)
    out = _sc_gather(x_flat, table)
    return out.reshape(BATCH, HIST, DIM)


# speedup vs baseline: 1.0204x; 1.0204x over previous
import functools

import jax
import jax.numpy as jnp
from jax import lax
from jax.experimental import pallas as pl
from jax.experimental.pallas import tpu as pltpu
from jax.experimental.pallas import tpu_sc as plsc

VOCAB = 1000000
DIM = 64
BATCH = 4096
HIST = 200

NC = 2
NS = 16
NW = NC * NS
TOTAL = BATCH * HIST
BPW = TOTAL // NW
CHUNK = 128
NCHUNK = BPW // CHUNK
G = 4
SB = G * CHUNK
NSUPER = NCHUNK // G

_mesh = plsc.VectorSubcoreMesh(core_axis_name="c", subcore_axis_name="s")


@functools.partial(
    pl.kernel,
    out_type=jax.ShapeDtypeStruct((TOTAL, DIM), jnp.float32),
    mesh=_mesh,
    scratch_types=[
        pltpu.VMEM((NCHUNK, CHUNK), jnp.int32),
        pltpu.VMEM((SB, DIM), jnp.float32),
        pltpu.VMEM((SB, DIM), jnp.float32),
        pltpu.SemaphoreType.DMA,
        pltpu.SemaphoreType.DMA,
        pltpu.SemaphoreType.DMA,
        pltpu.SemaphoreType.DMA,
    ],
    compiler_params=pltpu.CompilerParams(use_tc_tiling_on_sc=False),
)
def _sc_gather(x_hbm, table_hbm, out_hbm, idx_v, rows0, rows1,
               gsem0, gsem1, wsem0, wsem1):
    wid = lax.axis_index("s") * NC + lax.axis_index("c")
    base = wid * BPW
    pltpu.sync_copy(x_hbm.at[wid], idx_v)

    bufs = (rows0, rows1)
    gsems = (gsem0, gsem1)
    wsems = (wsem0, wsem1)

    def fire(s, slot):
        for g in range(G):
            pltpu.async_copy(
                table_hbm.at[idx_v.at[s * G + g]],
                bufs[slot].at[pl.ds(g * CHUNK, CHUNK)],
                gsems[slot],
            )

    def drain(s, slot):
        for g in range(G):
            pltpu.make_async_copy(
                table_hbm.at[idx_v.at[s * G + g]],
                bufs[slot].at[pl.ds(g * CHUNK, CHUNK)],
                gsems[slot],
            ).wait()

    def out_copy(s, slot):
        return pltpu.make_async_copy(
            bufs[slot], out_hbm.at[pl.ds(base + s * SB, SB)], wsems[slot]
        )

    fire(0, 0)

    def step(s, slot):
        @pl.when(s + 1 < NSUPER)
        def _():
            @pl.when(s >= 1)
            def _():
                out_copy(s - 1, 1 - slot).wait()

            fire(s + 1, 1 - slot)

        drain(s, slot)
        out_copy(s, slot).start()

    def body(s, _):
        @pl.when(lax.rem(s, 2) == 0)
        def _():
            step(s, 0)

        @pl.when(lax.rem(s, 2) == 1)
        def _():
            step(s, 1)

        return 0

    lax.fori_loop(0, NSUPER, body, 0, unroll=False)

    out_copy(NSUPER - 2, (NSUPER - 2) % 2).wait()
    out_copy(NSUPER - 1, (NSUPER - 1) % 2).wait()


def kernel(x, table):
    x_flat = x.astype(jnp.int32).reshape(NW, NCHUNK, CHUNK)
    out = _sc_gather(x_flat, table)
    return out.reshape(BATCH, HIST, DIM)
